# baseline (speedup 1.0000x reference)
import functools

import jax
import jax.numpy as jnp
from jax import lax
from jax.experimental import pallas as pl
from jax.experimental.pallas import tpu as pltpu
from jax.experimental.pallas import tpu_sc as plsc

B, N, K, D = 4, 10000, 32, 128
OUT, HID = 128, 256
NSUB = 16
C = 64
N_PAD = 10240
CH_PER_G = N_PAD // C
CH_PER_W = CH_PER_G // NSUB
SPC = CH_PER_W - 2
STG = 640
R = 1000
NC = N // R


def _sc_neighbor_sum(feat, idx_chunks):
    mesh = plsc.VectorSubcoreMesh(core_axis_name="c", subcore_axis_name="s")

    @functools.partial(
        pl.kernel,
        out_type=jax.ShapeDtypeStruct((B, N_PAD, D), jnp.float32),
        mesh=mesh,
        scratch_types=[
            pltpu.VMEM_SHARED((N, D), jnp.float32),
            pltpu.VMEM((2, K, C), jnp.int32),
            pltpu.VMEM((2, K, C), jnp.int32),
            pltpu.VMEM((C, D), jnp.float32),
            pltpu.VMEM((C, D), jnp.float32),
            pltpu.VMEM((C, D), jnp.float32),
            pltpu.VMEM((C, D), jnp.float32),
            pltpu.SemaphoreType.DMA,
            pltpu.SemaphoreType.DMA,
            pltpu.SemaphoreType.DMA,
            pltpu.SemaphoreType.DMA,
        ],
    )
    def sc_kernel(feat_hbm, idx_hbm, out_hbm, table, idx_v, idx_h,
                  acc0, acc1, acch0, acch1, sem0, sem1, semh0, semh1):
        cid = lax.axis_index("c")
        sid = lax.axis_index("s")
        accs = (acc0, acc1)
        sems = (sem0, sem1)

        def zero(acc):
            z = jnp.zeros((16,), jnp.float32)

            def zrow(r, c2):
                for i in range(D // 16):
                    acc[r, pl.ds(i * 16, 16)] = z
                return c2

            lax.fori_loop(0, C, zrow, 0)

        def fire(p, acc, sem):
            def fk(k, c2):
                pltpu.async_copy(table.at[idx_v.at[p, k]], acc, sem, add=True)
                return c2

            lax.fori_loop(0, K, fk, 0)

        def drain(acc, sem):
            def dk(k, c2):
                pltpu.make_async_copy(feat_hbm.at[0, pl.ds(0, C)], acc, sem).wait()
                return c2

            lax.fori_loop(0, K, dk, 0)

        for hh in range(2):
            b = 2 * cid + hh
            if hh == 1:
                plsc.subcore_barrier()

            @pl.when(sid < NSUB - 1)
            def _stage_full():
                pltpu.sync_copy(feat_hbm.at[b, pl.ds(sid * STG, STG)],
                                table.at[pl.ds(sid * STG, STG)])

            @pl.when(sid == NSUB - 1)
            def _stage_tail():
                pltpu.sync_copy(
                    feat_hbm.at[b, pl.ds((NSUB - 1) * STG, N - (NSUB - 1) * STG)],
                    table.at[pl.ds((NSUB - 1) * STG, N - (NSUB - 1) * STG)])

            pltpu.sync_copy(idx_hbm.at[b, sid * CH_PER_W], idx_v.at[0])
            pltpu.sync_copy(idx_hbm.at[b, sid * CH_PER_W + 1], idx_v.at[1])
            pltpu.sync_copy(idx_hbm.at[b, sid * CH_PER_W + SPC], idx_h.at[0])
            pltpu.sync_copy(idx_hbm.at[b, sid * CH_PER_W + SPC + 1], idx_h.at[1])
            zero(acc0)
            zero(acc1)
            zero(acch0)
            zero(acch1)
            plsc.subcore_barrier()

            def fh0(k, c2):
                pltpu.async_copy(feat_hbm.at[b].at[idx_h.at[0, k]], acch0,
                                 semh0, add=True)
                return c2

            lax.fori_loop(0, K, fh0, 0)

            def fh1(k, c2):
                pltpu.async_copy(feat_hbm.at[b].at[idx_h.at[1, k]], acch1,
                                 semh1, add=True)
                return c2

            lax.fori_loop(0, K, fh1, 0)

            fire(0, accs[0], sems[0])
            fire(1, accs[1], sems[1])
            for j in range(2, SPC + 2):
                p = j % 2
                drain(accs[p], sems[p])
                chunk = sid * CH_PER_W + (j - 2)
                pltpu.sync_copy(accs[p], out_hbm.at[b, pl.ds(chunk * C, C)])
                if j < SPC:
                    pltpu.sync_copy(idx_hbm.at[b, sid * CH_PER_W + j],
                                    idx_v.at[p])
                    zero(accs[p])
                    fire(p, accs[p], sems[p])

            drain(acch0, semh0)
            chunk_h = sid * CH_PER_W + SPC
            pltpu.sync_copy(acch0, out_hbm.at[b, pl.ds(chunk_h * C, C)])
            drain(acch1, semh1)
            pltpu.sync_copy(acch1, out_hbm.at[b, pl.ds((chunk_h + 1) * C, C)])

    return sc_kernel(feat, idx_chunks)


def _tc_dense(feat, nsum, actions, w_top, w_bot,
              f1w, f1b, f2w, f2b, f3w, f3b):
    def body(actions_ref, feat_ref, nsum_ref, wt_ref, wb_ref,
             f1w_ref, f1b_ref, f2w_ref, f2b_ref, f3w_ref, f3b_ref,
             out_ref, xbuf):
        i = pl.program_id(0)
        c = pl.program_id(1)

        @pl.when(jnp.logical_and(i == 0, c == 0))
        def _init():
            xbuf[...] = jnp.zeros_like(xbuf)

        f = feat_ref[0]
        s = nsum_ref[0]
        e = jnp.dot(f, wt_ref[...], preferred_element_type=jnp.float32)
        e = e + jnp.dot(s, wb_ref[...], preferred_element_type=jnp.float32)
        e = jnp.maximum(e, 0.0)

        part_sum = jnp.sum(e, axis=0, keepdims=True)
        act = actions_ref[i]
        rows = lax.broadcasted_iota(jnp.int32, (R, OUT), 0) + c * R
        mask = (rows == act).astype(jnp.float32)
        part_act = jnp.sum(e * mask, axis=0, keepdims=True)
        upd = jnp.concatenate([part_sum, part_act], axis=1)
        xbuf[pl.ds(i, 1), :] = xbuf[pl.ds(i, 1), :] + upd

        @pl.when(jnp.logical_and(i == B - 1, c == NC - 1))
        def _tail():
            scale = jnp.concatenate(
                [jnp.full((1, OUT), 1.0 / N, jnp.float32),
                 jnp.ones((1, OUT), jnp.float32)], axis=1)
            x = xbuf[...] * scale
            hh = jnp.dot(x, f1w_ref[...], preferred_element_type=jnp.float32)
            hh = jnp.maximum(hh + f1b_ref[...], 0.0)
            hh = jnp.dot(hh, f2w_ref[...], preferred_element_type=jnp.float32)
            hh = jnp.maximum(hh + f2b_ref[...], 0.0)
            o = jnp.dot(hh, f3w_ref[...], preferred_element_type=jnp.float32)
            out_ref[...] = o + f3b_ref[...]

    return pl.pallas_call(
        body,
        grid=(B, NC),
        in_specs=[
            pl.BlockSpec(memory_space=pltpu.SMEM),
            pl.BlockSpec((1, R, D), lambda i, c: (i, c, 0)),
            pl.BlockSpec((1, R, D), lambda i, c: (i, c, 0)),
            pl.BlockSpec((D, OUT), lambda i, c: (0, 0)),
            pl.BlockSpec((D, OUT), lambda i, c: (0, 0)),
            pl.BlockSpec((2 * OUT, HID), lambda i, c: (0, 0)),
            pl.BlockSpec((1, HID), lambda i, c: (0, 0)),
            pl.BlockSpec((HID, HID), lambda i, c: (0, 0)),
            pl.BlockSpec((1, HID), lambda i, c: (0, 0)),
            pl.BlockSpec((HID, OUT), lambda i, c: (0, 0)),
            pl.BlockSpec((1, OUT), lambda i, c: (0, 0)),
        ],
        out_specs=pl.BlockSpec((8, OUT), lambda i, c: (0, 0)),
        out_shape=jax.ShapeDtypeStruct((8, OUT), jnp.float32),
        scratch_shapes=[pltpu.VMEM((8, 2 * OUT), jnp.float32)],
    )(actions, feat, nsum, w_top, w_bot,
      f1w, f1b, f2w, f2b, f3w, f3b)


def kernel(actions, features, adj_lists, nodes, W_sage, fc1_w, fc1_b, fc2_w, fc2_b, fc3_w, fc3_b):
    del nodes
    adj = adj_lists.astype(jnp.int32)
    adj_pad = jnp.pad(adj, ((0, 0), (0, N_PAD - N), (0, 0)))
    idx_chunks = adj_pad.reshape(B, CH_PER_G, C, K).transpose(0, 1, 3, 2)

    w_top = W_sage[:D]
    w_bot = W_sage[D:] * (1.0 / K)
    f3w = jnp.pad(fc3_w, ((0, 0), (0, OUT - 1)))
    f3b = jnp.pad(fc3_b, (0, OUT - 1)).reshape(1, OUT)
    acts = actions.astype(jnp.int32)
    f1b2 = fc1_b.reshape(1, HID)
    f2b2 = fc2_b.reshape(1, HID)

    nsum = _sc_neighbor_sum(features, idx_chunks)

    out8 = _tc_dense(features, nsum, acts, w_top, w_bot,
                     fc1_w, f1b2, fc2_w, f2b2, f3w, f3b)
    return out8[:B, :1]


# speedup vs baseline: 2.1195x; 2.1195x over previous
import functools

import jax
import jax.numpy as jnp
from jax import lax
from jax.experimental import pallas as pl
from jax.experimental.pallas import tpu as pltpu
from jax.experimental.pallas import tpu_sc as plsc

B, N, K, D = 4, 10000, 32, 128
OUT, HID = 128, 256
NSUB = 16
C = 64
N_PAD = 10240
CH_PER_G = N_PAD // C
CH_PER_W = CH_PER_G // NSUB
SPC = CH_PER_W
STG = 640
R = 1000
NC = N // R


def _sc_neighbor_sum(feat, idx_chunks):
    mesh = plsc.VectorSubcoreMesh(core_axis_name="c", subcore_axis_name="s")

    @functools.partial(
        pl.kernel,
        out_type=jax.ShapeDtypeStruct((B, N_PAD, D), jnp.float32),
        mesh=mesh,
        scratch_types=[
            pltpu.VMEM_SHARED((N, D), jnp.float32),
            pltpu.VMEM((2, K, C), jnp.int32),
            pltpu.VMEM((C, D), jnp.float32),
            pltpu.VMEM((C, D), jnp.float32),
            pltpu.SemaphoreType.DMA,
            pltpu.SemaphoreType.DMA,
        ],
    )
    def sc_kernel(feat_hbm, idx_hbm, out_hbm, table, idx_v,
                  acc0, acc1, sem0, sem1):
        cid = lax.axis_index("c")
        sid = lax.axis_index("s")
        accs = (acc0, acc1)
        sems = (sem0, sem1)

        def zero(acc):
            z = jnp.zeros((16,), jnp.float32)

            def zrow(r, c2):
                for i in range(D // 16):
                    acc[r, pl.ds(i * 16, 16)] = z
                return c2

            lax.fori_loop(0, C, zrow, 0)

        def fire(p, acc, sem):
            def fk(k, c2):
                pltpu.async_copy(table.at[idx_v.at[p, k]], acc, sem, add=True)
                return c2

            lax.fori_loop(0, K, fk, 0)

        def drain(acc, sem):
            def dk(k, c2):
                pltpu.make_async_copy(feat_hbm.at[0, pl.ds(0, C)], acc, sem).wait()
                return c2

            lax.fori_loop(0, K, dk, 0)

        for hh in range(2):
            b = 2 * cid + hh
            if hh == 1:
                plsc.subcore_barrier()

            @pl.when(sid < NSUB - 1)
            def _stage_full():
                pltpu.sync_copy(feat_hbm.at[b, pl.ds(sid * STG, STG)],
                                table.at[pl.ds(sid * STG, STG)])

            @pl.when(sid == NSUB - 1)
            def _stage_tail():
                pltpu.sync_copy(
                    feat_hbm.at[b, pl.ds((NSUB - 1) * STG, N - (NSUB - 1) * STG)],
                    table.at[pl.ds((NSUB - 1) * STG, N - (NSUB - 1) * STG)])

            pltpu.sync_copy(idx_hbm.at[b, sid * CH_PER_W], idx_v.at[0])
            pltpu.sync_copy(idx_hbm.at[b, sid * CH_PER_W + 1], idx_v.at[1])
            zero(acc0)
            zero(acc1)
            plsc.subcore_barrier()

            fire(0, accs[0], sems[0])
            fire(1, accs[1], sems[1])
            for j in range(2, SPC + 2):
                p = j % 2
                drain(accs[p], sems[p])
                chunk = sid * CH_PER_W + (j - 2)
                pltpu.sync_copy(accs[p], out_hbm.at[b, pl.ds(chunk * C, C)])
                if j < SPC:
                    pltpu.sync_copy(idx_hbm.at[b, sid * CH_PER_W + j],
                                    idx_v.at[p])
                    zero(accs[p])
                    fire(p, accs[p], sems[p])


    return sc_kernel(feat, idx_chunks)


def _tc_dense(feat, nsum, actions, w_top, w_bot,
              f1w, f1b, f2w, f2b, f3w, f3b):
    def body(actions_ref, feat_ref, nsum_ref, wt_ref, wb_ref,
             f1w_ref, f1b_ref, f2w_ref, f2b_ref, f3w_ref, f3b_ref,
             out_ref, xbuf):
        i = pl.program_id(0)
        c = pl.program_id(1)

        @pl.when(jnp.logical_and(i == 0, c == 0))
        def _init():
            xbuf[...] = jnp.zeros_like(xbuf)

        f = feat_ref[0]
        s = nsum_ref[0]
        e = jnp.dot(f, wt_ref[...], preferred_element_type=jnp.float32)
        e = e + jnp.dot(s, wb_ref[...], preferred_element_type=jnp.float32)
        e = jnp.maximum(e, 0.0)

        part_sum = jnp.sum(e, axis=0, keepdims=True)
        act = actions_ref[i]
        rows = lax.broadcasted_iota(jnp.int32, (R, OUT), 0) + c * R
        mask = (rows == act).astype(jnp.float32)
        part_act = jnp.sum(e * mask, axis=0, keepdims=True)
        upd = jnp.concatenate([part_sum, part_act], axis=1)
        xbuf[pl.ds(i, 1), :] = xbuf[pl.ds(i, 1), :] + upd

        @pl.when(jnp.logical_and(i == B - 1, c == NC - 1))
        def _tail():
            scale = jnp.concatenate(
                [jnp.full((1, OUT), 1.0 / N, jnp.float32),
                 jnp.ones((1, OUT), jnp.float32)], axis=1)
            x = xbuf[...] * scale
            hh = jnp.dot(x, f1w_ref[...], preferred_element_type=jnp.float32)
            hh = jnp.maximum(hh + f1b_ref[...], 0.0)
            hh = jnp.dot(hh, f2w_ref[...], preferred_element_type=jnp.float32)
            hh = jnp.maximum(hh + f2b_ref[...], 0.0)
            o = jnp.dot(hh, f3w_ref[...], preferred_element_type=jnp.float32)
            out_ref[...] = o + f3b_ref[...]

    return pl.pallas_call(
        body,
        grid=(B, NC),
        in_specs=[
            pl.BlockSpec(memory_space=pltpu.SMEM),
            pl.BlockSpec((1, R, D), lambda i, c: (i, c, 0)),
            pl.BlockSpec((1, R, D), lambda i, c: (i, c, 0)),
            pl.BlockSpec((D, OUT), lambda i, c: (0, 0)),
            pl.BlockSpec((D, OUT), lambda i, c: (0, 0)),
            pl.BlockSpec((2 * OUT, HID), lambda i, c: (0, 0)),
            pl.BlockSpec((1, HID), lambda i, c: (0, 0)),
            pl.BlockSpec((HID, HID), lambda i, c: (0, 0)),
            pl.BlockSpec((1, HID), lambda i, c: (0, 0)),
            pl.BlockSpec((HID, OUT), lambda i, c: (0, 0)),
            pl.BlockSpec((1, OUT), lambda i, c: (0, 0)),
        ],
        out_specs=pl.BlockSpec((8, OUT), lambda i, c: (0, 0)),
        out_shape=jax.ShapeDtypeStruct((8, OUT), jnp.float32),
        scratch_shapes=[pltpu.VMEM((8, 2 * OUT), jnp.float32)],
    )(actions, feat, nsum, w_top, w_bot,
      f1w, f1b, f2w, f2b, f3w, f3b)


def kernel(actions, features, adj_lists, nodes, W_sage, fc1_w, fc1_b, fc2_w, fc2_b, fc3_w, fc3_b):
    del nodes
    adj = adj_lists.astype(jnp.int32)
    adj_pad = jnp.pad(adj, ((0, 0), (0, N_PAD - N), (0, 0)))
    idx_chunks = adj_pad.reshape(B, CH_PER_G, C, K).transpose(0, 1, 3, 2)

    w_top = W_sage[:D]
    w_bot = W_sage[D:] * (1.0 / K)
    f3w = jnp.pad(fc3_w, ((0, 0), (0, OUT - 1)))
    f3b = jnp.pad(fc3_b, (0, OUT - 1)).reshape(1, OUT)
    acts = actions.astype(jnp.int32)
    f1b2 = fc1_b.reshape(1, HID)
    f2b2 = fc2_b.reshape(1, HID)

    nsum = _sc_neighbor_sum(features, idx_chunks)

    out8 = _tc_dense(features, nsum, acts, w_top, w_bot,
                     fc1_w, f1b2, fc2_w, f2b2, f3w, f3b)
    return out8[:B, :1]
